# initial kernel scaffold (unmeasured)
import jax
import jax.numpy as jnp
from jax import lax
from jax.experimental import pallas as pl
from jax.experimental.pallas import tpu as pltpu

T = 4096
D = 1024
CH = 512
N_MAX = T // CH


def _body(cnt_ref, xs_ref, out_ref, send_sems, recv_sems):
    my_x = lax.axis_index("x")
    my_y = lax.axis_index("y")
    my_z = lax.axis_index("z")
    peer = (my_x, my_y, 1 - my_z)
    cnt0 = cnt_ref[0]

    is0 = my_z == 0
    send_count = jnp.where(is0, T - cnt0, cnt0)
    keep_count = T - send_count
    n_send = (send_count + CH - 1) // CH
    n_keep = (keep_count + CH - 1) // CH
    dst_shift = jnp.where(is0, -cnt0, T - cnt0)

    for i in range(N_MAX):
        src_start = jnp.where(
            is0,
            jnp.maximum(T - (i + 1) * CH, cnt0),
            jnp.minimum(i * CH, cnt0 - CH),
        )
        dst_start = src_start + dst_shift

        @pl.when(i < n_send)
        def _(i=i, src_start=src_start, dst_start=dst_start):
            rdma = pltpu.make_async_remote_copy(
                src_ref=xs_ref.at[pl.ds(src_start, CH), :],
                dst_ref=out_ref.at[pl.ds(dst_start, CH), :],
                send_sem=send_sems.at[i],
                recv_sem=recv_sems.at[i],
                device_id=peer,
                device_id_type=pl.DeviceIdType.MESH,
            )
            rdma.start()

    for i in range(N_MAX):
        start = jnp.where(
            is0,
            jnp.minimum(i * CH, cnt0 - CH),
            jnp.maximum(T - (i + 1) * CH, cnt0),
        )

        @pl.when(i < n_keep)
        def _(start=start):
            out_ref[pl.ds(start, CH), :] = xs_ref[pl.ds(start, CH), :]

    for i in range(N_MAX):

        @pl.when(i < n_send)
        def _(i=i):
            rdma = pltpu.make_async_remote_copy(
                src_ref=xs_ref.at[pl.ds(0, CH), :],
                dst_ref=out_ref.at[pl.ds(0, CH), :],
                send_sem=send_sems.at[i],
                recv_sem=recv_sems.at[i],
                device_id=peer,
                device_id_type=pl.DeviceIdType.MESH,
            )
            rdma.wait_recv()

    for i in range(N_MAX):

        @pl.when(i < n_send)
        def _(i=i):
            rdma = pltpu.make_async_remote_copy(
                src_ref=xs_ref.at[pl.ds(0, CH), :],
                dst_ref=out_ref.at[pl.ds(0, CH), :],
                send_sem=send_sems.at[i],
                recv_sem=recv_sems.at[i],
                device_id=peer,
                device_id_type=pl.DeviceIdType.MESH,
            )
            rdma.wait_send()


def kernel(x, dest):
    order = jnp.argsort(dest, stable=True)
    xs = jnp.take(x, order, axis=0)
    cnt0 = jnp.sum(dest == 0).astype(jnp.int32).reshape((1,))

    return pl.pallas_call(
        _body,
        out_shape=jax.ShapeDtypeStruct((T, D), jnp.float32),
        in_specs=[
            pl.BlockSpec(memory_space=pltpu.SMEM),
            pl.BlockSpec(memory_space=pltpu.VMEM),
        ],
        out_specs=pl.BlockSpec(memory_space=pltpu.VMEM),
        scratch_shapes=[
            pltpu.SemaphoreType.DMA((N_MAX,)),
            pltpu.SemaphoreType.DMA((N_MAX,)),
        ],
    )(cnt0, xs)


# baseline (device time: 274273 ns/iter reference)
import jax
import jax.numpy as jnp
from jax import lax
from jax.experimental import pallas as pl
from jax.experimental.pallas import tpu as pltpu

T = 4096
D = 1024
CH = 512
N_MAX = T // CH

ROW = (8, 128)


def _body(cnt_ref, xs_ref, out_ref, send_sems, recv_sems):
    my_x = lax.axis_index("x")
    my_y = lax.axis_index("y")
    my_z = lax.axis_index("z")
    peer = (my_x, my_y, 1 - my_z)
    cnt0 = cnt_ref[0]

    is0 = my_z == 0
    send_count = jnp.where(is0, T - cnt0, cnt0)
    keep_count = T - send_count
    n_send = (send_count + CH - 1) // CH
    n_keep = (keep_count + CH - 1) // CH
    dst_shift = jnp.where(is0, -cnt0, T - cnt0)

    for i in range(N_MAX):
        src_start = jnp.where(
            is0,
            jnp.maximum(T - (i + 1) * CH, cnt0),
            jnp.minimum(i * CH, cnt0 - CH),
        )
        dst_start = src_start + dst_shift

        @pl.when(i < n_send)
        def _(i=i, src_start=src_start, dst_start=dst_start):
            rdma = pltpu.make_async_remote_copy(
                src_ref=xs_ref.at[pl.ds(src_start, CH)],
                dst_ref=out_ref.at[pl.ds(dst_start, CH)],
                send_sem=send_sems.at[i],
                recv_sem=recv_sems.at[i],
                device_id=peer,
                device_id_type=pl.DeviceIdType.MESH,
            )
            rdma.start()

    for i in range(N_MAX):
        start = jnp.where(
            is0,
            jnp.minimum(i * CH, cnt0 - CH),
            jnp.maximum(T - (i + 1) * CH, cnt0),
        )

        @pl.when(i < n_keep)
        def _(start=start):
            out_ref[pl.ds(start, CH)] = xs_ref[pl.ds(start, CH)]

    for i in range(N_MAX):

        @pl.when(i < n_send)
        def _(i=i):
            rdma = pltpu.make_async_remote_copy(
                src_ref=xs_ref.at[pl.ds(0, CH)],
                dst_ref=out_ref.at[pl.ds(0, CH)],
                send_sem=send_sems.at[i],
                recv_sem=recv_sems.at[i],
                device_id=peer,
                device_id_type=pl.DeviceIdType.MESH,
            )
            rdma.wait_recv()

    for i in range(N_MAX):

        @pl.when(i < n_send)
        def _(i=i):
            rdma = pltpu.make_async_remote_copy(
                src_ref=xs_ref.at[pl.ds(0, CH)],
                dst_ref=out_ref.at[pl.ds(0, CH)],
                send_sem=send_sems.at[i],
                recv_sem=recv_sems.at[i],
                device_id=peer,
                device_id_type=pl.DeviceIdType.MESH,
            )
            rdma.wait_send()


def kernel(x, dest):
    order = jnp.argsort(dest, stable=True)
    xs = jnp.take(x, order, axis=0).reshape(T, *ROW)
    cnt0 = jnp.sum(dest == 0).astype(jnp.int32).reshape((1,))

    out = pl.pallas_call(
        _body,
        out_shape=jax.ShapeDtypeStruct((T, *ROW), jnp.float32),
        in_specs=[
            pl.BlockSpec(memory_space=pltpu.SMEM),
            pl.BlockSpec(memory_space=pltpu.VMEM),
        ],
        out_specs=pl.BlockSpec(memory_space=pltpu.VMEM),
        scratch_shapes=[
            pltpu.SemaphoreType.DMA((N_MAX,)),
            pltpu.SemaphoreType.DMA((N_MAX,)),
        ],
    )(cnt0, xs)
    return out.reshape(T, D)


# device time: 143004 ns/iter; 1.9179x vs baseline; 1.9179x over previous
import jax
import jax.numpy as jnp
from jax import lax
from jax.experimental import pallas as pl
from jax.experimental.pallas import tpu as pltpu

T = 4096
D = 1024
CH = 512
N_MAX = T // CH
ROW = (8, 128)


def _body(cnt_ref, order_ref, x_ref, out_ref, xs_ref, gat_sem, keep_sem,
          send_sems, recv_sems):
    my_x = lax.axis_index("x")
    my_y = lax.axis_index("y")
    my_z = lax.axis_index("z")
    peer = (my_x, my_y, 1 - my_z)
    cnt0 = cnt_ref[0]

    is0 = my_z == 0
    send_count = jnp.where(is0, T - cnt0, cnt0)
    keep_count = T - send_count
    n_send = (send_count + CH - 1) // CH
    n_keep = (keep_count + CH - 1) // CH
    dst_shift = jnp.where(is0, -cnt0, T - cnt0)

    def gather_rows(dst_ref, start, sem):
        def one(k, _):
            j = start + k
            pltpu.make_async_copy(
                x_ref.at[order_ref[j]], dst_ref.at[j], sem
            ).start()
            return 0

        lax.fori_loop(0, CH, one, 0)

    def chunk_bytes_wait(sem, dst_ref):
        pltpu.make_async_copy(
            x_ref.at[pl.ds(0, CH)], dst_ref.at[pl.ds(0, CH)], sem
        ).wait()

    for i in range(N_MAX):
        src_start = jnp.where(
            is0,
            jnp.maximum(T - (i + 1) * CH, cnt0),
            jnp.minimum(i * CH, cnt0 - CH),
        )
        dst_start = src_start + dst_shift

        @pl.when(i < n_send)
        def _(i=i, src_start=src_start, dst_start=dst_start):
            gather_rows(xs_ref, src_start, gat_sem)
            chunk_bytes_wait(gat_sem, xs_ref)
            rdma = pltpu.make_async_remote_copy(
                src_ref=xs_ref.at[pl.ds(src_start, CH)],
                dst_ref=out_ref.at[pl.ds(dst_start, CH)],
                send_sem=send_sems.at[i],
                recv_sem=recv_sems.at[i],
                device_id=peer,
                device_id_type=pl.DeviceIdType.MESH,
            )
            rdma.start()

    for i in range(N_MAX):
        start = jnp.where(
            is0,
            jnp.minimum(i * CH, cnt0 - CH),
            jnp.maximum(T - (i + 1) * CH, cnt0),
        )

        @pl.when(i < n_keep)
        def _(start=start):
            gather_rows(out_ref, start, keep_sem)

    for i in range(N_MAX):

        @pl.when(i < n_keep)
        def _():
            chunk_bytes_wait(keep_sem, out_ref)

    for i in range(N_MAX):

        @pl.when(i < n_send)
        def _(i=i):
            rdma = pltpu.make_async_remote_copy(
                src_ref=xs_ref.at[pl.ds(0, CH)],
                dst_ref=out_ref.at[pl.ds(0, CH)],
                send_sem=send_sems.at[i],
                recv_sem=recv_sems.at[i],
                device_id=peer,
                device_id_type=pl.DeviceIdType.MESH,
            )
            rdma.wait_recv()

    for i in range(N_MAX):

        @pl.when(i < n_send)
        def _(i=i):
            rdma = pltpu.make_async_remote_copy(
                src_ref=xs_ref.at[pl.ds(0, CH)],
                dst_ref=out_ref.at[pl.ds(0, CH)],
                send_sem=send_sems.at[i],
                recv_sem=recv_sems.at[i],
                device_id=peer,
                device_id_type=pl.DeviceIdType.MESH,
            )
            rdma.wait_send()


def kernel(x, dest):
    order = jnp.argsort(dest, stable=True).astype(jnp.int32)
    cnt0 = jnp.sum(dest == 0).astype(jnp.int32).reshape((1,))

    out = pl.pallas_call(
        _body,
        out_shape=jax.ShapeDtypeStruct((T, *ROW), jnp.float32),
        in_specs=[
            pl.BlockSpec(memory_space=pltpu.SMEM),
            pl.BlockSpec(memory_space=pltpu.SMEM),
            pl.BlockSpec(memory_space=pltpu.VMEM),
        ],
        out_specs=pl.BlockSpec(memory_space=pltpu.VMEM),
        scratch_shapes=[
            pltpu.VMEM((T, *ROW), jnp.float32),
            pltpu.SemaphoreType.DMA,
            pltpu.SemaphoreType.DMA,
            pltpu.SemaphoreType.DMA((N_MAX,)),
            pltpu.SemaphoreType.DMA((N_MAX,)),
        ],
    )(cnt0, order, x.reshape(T, *ROW))
    return out.reshape(T, D)
